# Initial kernel scaffold; baseline (speedup 1.0000x reference)
import functools

import jax
import jax.numpy as jnp
from jax import lax
from jax.experimental import pallas as pl
from jax.experimental.pallas import tpu as pltpu
from jax.experimental.pallas import tpu_sc as plsc

N = 10000
E = 320000
D = 128

NC = 2
NS = 16
NW = NC * NS
EPW = E // NW
C = 80
NCHUNK = EPW // C
RPS = N // NS
ZR = 25

_mesh = plsc.VectorSubcoreMesh(core_axis_name="c", subcore_axis_name="s")


@functools.partial(
    pl.kernel,
    mesh=_mesh,
    out_type=[
        jax.ShapeDtypeStruct((NC * N, D), jnp.float32),
        jax.ShapeDtypeStruct((NC * N, 16), jnp.float32),
    ],
    scratch_types=[
        pltpu.VMEM((C,), jnp.int32),
        pltpu.VMEM((C,), jnp.int32),
        pltpu.VMEM((C, D), jnp.float32),
        pltpu.VMEM((C, 16), jnp.float32),
        pltpu.VMEM((25, D), jnp.float32),
        pltpu.VMEM((25, 16), jnp.float32),
        pltpu.VMEM_SHARED((N, D), jnp.float32),
        pltpu.VMEM_SHARED((N, 16), jnp.float32),
        pltpu.SemaphoreType.DMA,
    ],
)
def _sc_segsum(src_hbm, dst_hbm, feat_hbm, out_sum, out_deg,
               src_v, dst_v, rows_v, ones_v, zrow_v, zdeg_v,
               acc_s, deg_s, sem):
    cid = lax.axis_index("c")
    sid = lax.axis_index("s")
    wid = cid * NS + sid

    zero16 = jnp.zeros((16,), jnp.float32)
    one16 = jnp.ones((16,), jnp.float32)

    def _fill(i, carry):
        for c in range(D // 16):
            zrow_v[i, pl.ds(c * 16, 16)] = zero16
        zdeg_v[i, :] = zero16
        return carry
    lax.fori_loop(0, ZR, _fill, 0)

    def _fill_ones(i, carry):
        ones_v[i, :] = one16
        return carry
    lax.fori_loop(0, C, _fill_ones, 0)

    def _zero(j, carry):
        row = sid * RPS + j * ZR
        pltpu.sync_copy(zrow_v, acc_s.at[pl.ds(row, ZR)])
        pltpu.sync_copy(zdeg_v, deg_s.at[pl.ds(row, ZR)])
        return carry
    lax.fori_loop(0, RPS // ZR, _zero, 0)

    plsc.subcore_barrier()

    def _chunk(i, carry):
        base = wid * EPW + i * C
        pltpu.sync_copy(src_hbm.at[pl.ds(base, C)], src_v)
        pltpu.sync_copy(dst_hbm.at[pl.ds(base, C)], dst_v)
        pltpu.async_copy(feat_hbm.at[src_v], rows_v, sem).wait()
        pltpu.sync_copy(rows_v, acc_s.at[dst_v], add=True)
        pltpu.sync_copy(ones_v, deg_s.at[dst_v], add=True)
        return carry
    lax.fori_loop(0, NCHUNK, _chunk, 0)

    plsc.subcore_barrier()

    row0 = sid * RPS
    pltpu.sync_copy(acc_s.at[pl.ds(row0, RPS)],
                    out_sum.at[pl.ds(cid * N + row0, RPS)])
    pltpu.sync_copy(deg_s.at[pl.ds(row0, RPS)],
                    out_deg.at[pl.ds(cid * N + row0, RPS)])


BLK = 1000


def _tc_body(feat_r, ps_r, pd_r, wn_r, ws_r, b_r, out_r):
    s = ps_r[0] + ps_r[1]
    d = pd_r[0, :, 0:1] + pd_r[1, :, 0:1]
    h = s / jnp.maximum(d, 1.0)
    a = lax.dot_general(feat_r[...], ws_r[...], (((1,), (1,)), ((), ())),
                        preferred_element_type=jnp.float32)
    m = lax.dot_general(h, wn_r[...], (((1,), (1,)), ((), ())),
                        preferred_element_type=jnp.float32)
    out_r[...] = a + m + b_r[0]


BLK_GRID = N // BLK


def kernel(feat, edge_index, W_neigh, W_self, b_self):
    src = edge_index[0].astype(jnp.int32)
    dst = edge_index[1].astype(jnp.int32)

    psum_flat, pdeg_flat = _sc_segsum(src, dst, feat)
    psum = psum_flat.reshape(NC, N, D)
    pdeg = pdeg_flat.reshape(NC, N, 16)

    out = pl.pallas_call(
        _tc_body,
        grid=(BLK_GRID,),
        in_specs=[
            pl.BlockSpec((BLK, D), lambda i: (i, 0)),
            pl.BlockSpec((NC, BLK, D), lambda i: (0, i, 0)),
            pl.BlockSpec((NC, BLK, 16), lambda i: (0, i, 0)),
            pl.BlockSpec((D, D), lambda i: (0, 0)),
            pl.BlockSpec((D, D), lambda i: (0, 0)),
            pl.BlockSpec((1, D), lambda i: (0, 0)),
        ],
        out_specs=pl.BlockSpec((BLK, D), lambda i: (i, 0)),
        out_shape=jax.ShapeDtypeStruct((N, D), jnp.float32),
    )(feat, psum, pdeg, W_neigh, W_self, b_self.reshape(1, D))
    return out


# speedup vs baseline: 6.2673x; 6.2673x over previous
import functools

import jax
import jax.numpy as jnp
from jax import lax
from jax.experimental import pallas as pl
from jax.experimental.pallas import tpu as pltpu
from jax.experimental.pallas import tpu_sc as plsc

N = 10000
E = 320000
D = 128
NP = 10240

NC = 2
NS = 16
NW = NC * NS
EPW = E // NW
C = 80
NCHUNK = EPW // C
RPS = NP // NS
ZR = 32

_mesh = plsc.VectorSubcoreMesh(core_axis_name="c", subcore_axis_name="s")


@functools.partial(
    pl.kernel,
    mesh=_mesh,
    out_type=[
        jax.ShapeDtypeStruct((NC * NP, D), jnp.float32),
        jax.ShapeDtypeStruct((NW * NP,), jnp.float32),
    ],
    scratch_types=[
        pltpu.VMEM((C,), jnp.int32),
        pltpu.VMEM((C,), jnp.int32),
        pltpu.VMEM((C, D), jnp.float32),
        pltpu.VMEM((ZR, D), jnp.float32),
        pltpu.VMEM((ZR, D), jnp.float32),
        pltpu.VMEM((NP,), jnp.float32),
        pltpu.VMEM_SHARED((NP, D), jnp.float32),
        pltpu.SemaphoreType.DMA,
    ],
    compiler_params=pltpu.CompilerParams(needs_layout_passes=False),
)
def _sc_segsum(src_hbm, dst_hbm, feat_hbm, out_sum, out_deg,
               src_v, dst_v, rows_v, zrow_v, wrow_v, deg_v, acc_s, sem):
    cid = lax.axis_index("c")
    sid = lax.axis_index("s")
    wid = cid * NS + sid

    zero16 = jnp.zeros((16,), jnp.float32)
    one16 = jnp.ones((16,), jnp.float32)

    def _fill(i, carry):
        for c in range(D // 16):
            zrow_v[i, pl.ds(c * 16, 16)] = zero16
        return carry
    lax.fori_loop(0, ZR, _fill, 0)

    def _zdeg(k, carry):
        deg_v[pl.ds(k * 16, 16)] = zero16
        return carry
    lax.fori_loop(0, NP // 16, _zdeg, 0)

    def _zero(j, carry):
        pltpu.sync_copy(zrow_v, acc_s.at[pl.ds(sid * RPS + j * ZR, ZR)])
        return carry
    lax.fori_loop(0, RPS // ZR, _zero, 0)

    plsc.subcore_barrier()

    def _chunk(i, carry):
        base = wid * EPW + i * C
        pltpu.sync_copy(src_hbm.at[pl.ds(base, C)], src_v)
        pltpu.sync_copy(dst_hbm.at[pl.ds(base, C)], dst_v)
        pltpu.async_copy(feat_hbm.at[src_v], rows_v, sem).wait()
        pltpu.sync_copy(rows_v, acc_s.at[dst_v], add=True)
        for k in range(C // 16):
            idx16 = dst_v[pl.ds(k * 16, 16)]
            plsc.addupdate_scatter(deg_v, [idx16], one16)
        return carry
    lax.fori_loop(0, NCHUNK, _chunk, 0)

    plsc.subcore_barrier()

    def _writeout(j, carry):
        row = sid * RPS + j * ZR
        pltpu.sync_copy(acc_s.at[pl.ds(row, ZR)], wrow_v)
        pltpu.sync_copy(wrow_v, out_sum.at[pl.ds(cid * NP + row, ZR)])
        return carry
    lax.fori_loop(0, RPS // ZR, _writeout, 0)
    pltpu.sync_copy(deg_v, out_deg.at[pl.ds(wid * NP, NP)])


BLK = 640
BLK_GRID = NP // BLK


def _tc_body(feat_r, ps_r, pd_r, wn_r, ws_r, b_r, out_r):
    s = ps_r[0] + ps_r[1]
    d = jnp.sum(pd_r[...], axis=0)[:, None]
    h = s / jnp.maximum(d, 1.0)
    a = lax.dot_general(feat_r[...], ws_r[...], (((1,), (1,)), ((), ())),
                        preferred_element_type=jnp.float32)
    m = lax.dot_general(h, wn_r[...], (((1,), (1,)), ((), ())),
                        preferred_element_type=jnp.float32)
    out_r[...] = a + m + b_r[0]


def kernel(feat, edge_index, W_neigh, W_self, b_self):
    src = edge_index[0].astype(jnp.int32)
    dst = edge_index[1].astype(jnp.int32)

    psum_flat, pdeg_flat = _sc_segsum(src, dst, feat)
    psum = psum_flat.reshape(NC, NP, D)
    pdeg = pdeg_flat.reshape(NW, NP)

    out = pl.pallas_call(
        _tc_body,
        grid=(BLK_GRID,),
        in_specs=[
            pl.BlockSpec((BLK, D), lambda i: (i, 0)),
            pl.BlockSpec((NC, BLK, D), lambda i: (0, i, 0)),
            pl.BlockSpec((NW, BLK), lambda i: (0, i)),
            pl.BlockSpec((D, D), lambda i: (0, 0)),
            pl.BlockSpec((D, D), lambda i: (0, 0)),
            pl.BlockSpec((1, D), lambda i: (0, 0)),
        ],
        out_specs=pl.BlockSpec((BLK, D), lambda i: (i, 0)),
        out_shape=jax.ShapeDtypeStruct((N, D), jnp.float32),
    )(feat, psum, pdeg, W_neigh, W_self, b_self.reshape(1, D))
    return out


# speedup vs baseline: 9.9872x; 1.5935x over previous
import functools

import jax
import jax.numpy as jnp
from jax import lax
from jax.experimental import pallas as pl
from jax.experimental.pallas import tpu as pltpu
from jax.experimental.pallas import tpu_sc as plsc

N = 10000
E = 320000
D = 128
NP = 10240

NC = 2
NS = 16
NW = NC * NS
EPW = E // NW
C = 80
NB = 3
NCHUNK = EPW // C
NSTEP = NCHUNK // NB
ZRS = 16
RPS = NP // NS
ZR = 16

_mesh = plsc.VectorSubcoreMesh(core_axis_name="c", subcore_axis_name="s")


@functools.partial(
    pl.kernel,
    mesh=_mesh,
    out_type=[
        jax.ShapeDtypeStruct((NC * NP, D), jnp.float32),
        jax.ShapeDtypeStruct((NW * NP,), jnp.float32),
    ],
    scratch_types=[
        [pltpu.VMEM((C,), jnp.int32) for _ in range(NB)],
        [pltpu.VMEM((C,), jnp.int32) for _ in range(NB)],
        [pltpu.VMEM((C, D), jnp.float32) for _ in range(NB)],
        pltpu.VMEM((ZR, D), jnp.float32),
        pltpu.VMEM((ZR, D), jnp.float32),
        pltpu.VMEM((NP,), jnp.float32),
        pltpu.VMEM_SHARED((NP, D), jnp.float32),
        [pltpu.SemaphoreType.DMA for _ in range(NB)],
        [pltpu.SemaphoreType.DMA for _ in range(NB)],
    ],
    compiler_params=pltpu.CompilerParams(needs_layout_passes=False),
)
def _sc_segsum(src_hbm, dst_hbm, feat_hbm, out_sum, out_deg,
               sidx_v, didx_v, rows_v, zrow_v, wrow_v, deg_v, acc_s,
               gsem, isem):
    cid = lax.axis_index("c")
    sid = lax.axis_index("s")
    wid = cid * NS + sid

    zero16 = jnp.zeros((16,), jnp.float32)
    one16 = jnp.ones((16,), jnp.float32)

    def _fill(i, carry):
        for c in range(D // 16):
            zrow_v[i, pl.ds(c * 16, 16)] = zero16
        return carry
    lax.fori_loop(0, ZR, _fill, 0)

    def _zdeg(k, carry):
        deg_v[pl.ds(k * 16, 16)] = zero16
        return carry
    lax.fori_loop(0, NP // 16, _zdeg, 0)

    def _zero(j, carry):
        pltpu.sync_copy(zrow_v, acc_s.at[pl.ds(sid * RPS + j * ZR, ZR)])
        return carry
    lax.fori_loop(0, RPS // ZR, _zero, 0)

    plsc.subcore_barrier()

    def _burst(c0, nb):
        idesc = []
        for b in range(nb):
            base = wid * EPW + (c0 + b) * C
            idesc.append((
                pltpu.async_copy(src_hbm.at[pl.ds(base, C)], sidx_v[b], isem[b]),
                pltpu.async_copy(dst_hbm.at[pl.ds(base, C)], didx_v[b], isem[b]),
            ))
        gdesc = []
        for b in range(nb):
            idesc[b][0].wait()
            idesc[b][1].wait()
            gdesc.append(pltpu.async_copy(
                feat_hbm.at[sidx_v[b]], rows_v[b], gsem[b]))
        for b in range(nb):
            gdesc[b].wait()
            for k in range(C // 16):
                idx16 = didx_v[b][pl.ds(k * 16, 16)]
                plsc.addupdate_scatter(deg_v, [idx16], one16)
            pltpu.sync_copy(rows_v[b], acc_s.at[didx_v[b]], add=True)

    def _step(j, carry):
        _burst(NB * j, NB)
        return carry
    lax.fori_loop(0, NSTEP, _step, 0)
    _burst(NSTEP * NB, NCHUNK - NSTEP * NB)

    plsc.subcore_barrier()

    def _writeout(j, carry):
        row = sid * RPS + j * ZR
        pltpu.sync_copy(acc_s.at[pl.ds(row, ZR)], wrow_v)
        pltpu.sync_copy(wrow_v, out_sum.at[pl.ds(cid * NP + row, ZR)])
        return carry
    lax.fori_loop(0, RPS // ZR, _writeout, 0)
    pltpu.sync_copy(deg_v, out_deg.at[pl.ds(wid * NP, NP)])


BLK = 640
BLK_GRID = NP // BLK


def _tc_body(feat_r, ps_r, pd_r, wn_r, ws_r, b_r, out_r):
    s = ps_r[0] + ps_r[1]
    d = jnp.sum(pd_r[...], axis=0)[:, None]
    h = s / jnp.maximum(d, 1.0)
    a = lax.dot_general(feat_r[...], ws_r[...], (((1,), (1,)), ((), ())),
                        preferred_element_type=jnp.float32)
    m = lax.dot_general(h, wn_r[...], (((1,), (1,)), ((), ())),
                        preferred_element_type=jnp.float32)
    out_r[...] = a + m + b_r[0]


def kernel(feat, edge_index, W_neigh, W_self, b_self):
    src = edge_index[0].astype(jnp.int32)
    dst = edge_index[1].astype(jnp.int32)

    psum_flat, pdeg_flat = _sc_segsum(src, dst, feat)
    psum = psum_flat.reshape(NC, NP, D)
    pdeg = pdeg_flat.reshape(NW, NP)

    out = pl.pallas_call(
        _tc_body,
        grid=(BLK_GRID,),
        in_specs=[
            pl.BlockSpec((BLK, D), lambda i: (i, 0)),
            pl.BlockSpec((NC, BLK, D), lambda i: (0, i, 0)),
            pl.BlockSpec((NW, BLK), lambda i: (0, i)),
            pl.BlockSpec((D, D), lambda i: (0, 0)),
            pl.BlockSpec((D, D), lambda i: (0, 0)),
            pl.BlockSpec((1, D), lambda i: (0, 0)),
        ],
        out_specs=pl.BlockSpec((BLK, D), lambda i: (i, 0)),
        out_shape=jax.ShapeDtypeStruct((N, D), jnp.float32),
    )(feat, psum, pdeg, W_neigh, W_self, b_self.reshape(1, D))
    return out


# speedup vs baseline: 10.2791x; 1.0292x over previous
import functools

import jax
import jax.numpy as jnp
from jax import lax
from jax.experimental import pallas as pl
from jax.experimental.pallas import tpu as pltpu
from jax.experimental.pallas import tpu_sc as plsc

N = 10000
E = 320000
D = 128
NP = 10240

NC = 2
NS = 16
NW = NC * NS
EPW = E // NW
C = 80
NB = 3
NCHUNK = EPW // C
NSTEP = NCHUNK // NB
ZRS = 16
RPS = NP // NS
ZR = 16

_mesh = plsc.VectorSubcoreMesh(core_axis_name="c", subcore_axis_name="s")


@functools.partial(
    pl.kernel,
    mesh=_mesh,
    out_type=[
        jax.ShapeDtypeStruct((NC * NP, D), jnp.float32),
        jax.ShapeDtypeStruct((NW * NP,), jnp.float32),
    ],
    scratch_types=[
        [pltpu.VMEM((C,), jnp.int32) for _ in range(NB)],
        [pltpu.VMEM((C,), jnp.int32) for _ in range(NB)],
        [pltpu.VMEM((C, D), jnp.float32) for _ in range(NB)],
        pltpu.VMEM((ZR, D), jnp.float32),
        pltpu.VMEM((ZR, D), jnp.float32),
        pltpu.VMEM((NP,), jnp.float32),
        pltpu.VMEM_SHARED((NP, D), jnp.float32),
        [pltpu.SemaphoreType.DMA for _ in range(NB)],
        [pltpu.SemaphoreType.DMA for _ in range(NB)],
        [pltpu.SemaphoreType.DMA for _ in range(NB)],
    ],
    compiler_params=pltpu.CompilerParams(needs_layout_passes=False),
)
def _sc_segsum(src_hbm, dst_hbm, feat_hbm, out_sum, out_deg,
               sidx_v, didx_v, rows_v, zrow_v, wrow_v, deg_v, acc_s,
               gsem, isem, ssem):
    cid = lax.axis_index("c")
    sid = lax.axis_index("s")
    wid = cid * NS + sid

    zero16 = jnp.zeros((16,), jnp.float32)
    one16 = jnp.ones((16,), jnp.float32)

    def _fill(i, carry):
        for c in range(D // 16):
            zrow_v[i, pl.ds(c * 16, 16)] = zero16
        return carry
    lax.fori_loop(0, ZR, _fill, 0)

    def _zdeg(k, carry):
        deg_v[pl.ds(k * 16, 16)] = zero16
        return carry
    lax.fori_loop(0, NP // 16, _zdeg, 0)

    def _zero(j, carry):
        pltpu.sync_copy(zrow_v, acc_s.at[pl.ds(sid * RPS + j * ZR, ZR)])
        return carry
    lax.fori_loop(0, RPS // ZR, _zero, 0)

    plsc.subcore_barrier()

    def _wait_scatter(b):
        pltpu.make_async_copy(rows_v[b], acc_s.at[didx_v[b]], ssem[b]).wait()

    def _burst(c0, nb, waits):
        for b in waits:
            _wait_scatter(b)
        idesc = []
        for b in range(nb):
            base = wid * EPW + (c0 + b) * C
            idesc.append((
                pltpu.async_copy(src_hbm.at[pl.ds(base, C)], sidx_v[b], isem[b]),
                pltpu.async_copy(dst_hbm.at[pl.ds(base, C)], didx_v[b], isem[b]),
            ))
        gdesc = []
        for b in range(nb):
            idesc[b][0].wait()
            idesc[b][1].wait()
            gdesc.append(pltpu.async_copy(
                feat_hbm.at[sidx_v[b]], rows_v[b], gsem[b]))
        for b in range(nb):
            gdesc[b].wait()
            for k in range(C // 16):
                idx16 = didx_v[b][pl.ds(k * 16, 16)]
                plsc.addupdate_scatter(deg_v, [idx16], one16)
            pltpu.async_copy(rows_v[b], acc_s.at[didx_v[b]], ssem[b], add=True)

    TAIL = NCHUNK - NSTEP * NB
    _burst(0, NB, [])

    def _step(j, carry):
        _burst(NB * (j + 1), NB, range(NB))
        return carry
    lax.fori_loop(0, NSTEP - 1, _step, 0)
    _burst(NSTEP * NB, TAIL, range(NB))
    for b in range(TAIL):
        _wait_scatter(b)

    plsc.subcore_barrier()

    def _writeout(j, carry):
        row = sid * RPS + j * ZR
        pltpu.sync_copy(acc_s.at[pl.ds(row, ZR)], wrow_v)
        pltpu.sync_copy(wrow_v, out_sum.at[pl.ds(cid * NP + row, ZR)])
        return carry
    lax.fori_loop(0, RPS // ZR, _writeout, 0)
    pltpu.sync_copy(deg_v, out_deg.at[pl.ds(wid * NP, NP)])


BLK = 640
BLK_GRID = NP // BLK


def _tc_body(feat_r, ps_r, pd_r, wn_r, ws_r, b_r, out_r):
    s = ps_r[0] + ps_r[1]
    d = jnp.sum(pd_r[...], axis=0)[:, None]
    h = s / jnp.maximum(d, 1.0)
    a = lax.dot_general(feat_r[...], ws_r[...], (((1,), (1,)), ((), ())),
                        preferred_element_type=jnp.float32)
    m = lax.dot_general(h, wn_r[...], (((1,), (1,)), ((), ())),
                        preferred_element_type=jnp.float32)
    out_r[...] = a + m + b_r[0]


def kernel(feat, edge_index, W_neigh, W_self, b_self):
    src = edge_index[0].astype(jnp.int32)
    dst = edge_index[1].astype(jnp.int32)

    psum_flat, pdeg_flat = _sc_segsum(src, dst, feat)
    psum = psum_flat.reshape(NC, NP, D)
    pdeg = pdeg_flat.reshape(NW, NP)

    out = pl.pallas_call(
        _tc_body,
        grid=(BLK_GRID,),
        in_specs=[
            pl.BlockSpec((BLK, D), lambda i: (i, 0)),
            pl.BlockSpec((NC, BLK, D), lambda i: (0, i, 0)),
            pl.BlockSpec((NW, BLK), lambda i: (0, i)),
            pl.BlockSpec((D, D), lambda i: (0, 0)),
            pl.BlockSpec((D, D), lambda i: (0, 0)),
            pl.BlockSpec((1, D), lambda i: (0, 0)),
        ],
        out_specs=pl.BlockSpec((BLK, D), lambda i: (i, 0)),
        out_shape=jax.ShapeDtypeStruct((N, D), jnp.float32),
    )(feat, psum, pdeg, W_neigh, W_self, b_self.reshape(1, D))
    return out


# speedup vs baseline: 10.3017x; 1.0022x over previous
import functools

import jax
import jax.numpy as jnp
from jax import lax
from jax.experimental import pallas as pl
from jax.experimental.pallas import tpu as pltpu
from jax.experimental.pallas import tpu_sc as plsc

N = 10000
E = 320000
D = 128
NP = 10240

NC = 2
NS = 16
NW = NC * NS
EPW = E // NW
C = 128
NB = 2
NFULL = EPW // C
NSTEP = NFULL // NB
CT = EPW - NFULL * C
RPS = NP // NS
ZR = 16

_mesh = plsc.VectorSubcoreMesh(core_axis_name="c", subcore_axis_name="s")


@functools.partial(
    pl.kernel,
    mesh=_mesh,
    out_type=[
        jax.ShapeDtypeStruct((NC * NP, D), jnp.float32),
        jax.ShapeDtypeStruct((NW * NP,), jnp.float32),
    ],
    scratch_types=[
        [pltpu.VMEM((C,), jnp.int32) for _ in range(NB)],
        [pltpu.VMEM((C,), jnp.int32) for _ in range(NB)],
        [pltpu.VMEM((C, D), jnp.float32) for _ in range(NB)],
        pltpu.VMEM((CT,), jnp.int32),
        pltpu.VMEM((CT,), jnp.int32),
        pltpu.VMEM((CT, D), jnp.float32),
        pltpu.VMEM((NP,), jnp.float32),
        pltpu.VMEM_SHARED((NP, D), jnp.float32),
        [pltpu.SemaphoreType.DMA for _ in range(NB)],
        [pltpu.SemaphoreType.DMA for _ in range(NB)],
        [pltpu.SemaphoreType.DMA for _ in range(NB)],
    ],
    compiler_params=pltpu.CompilerParams(needs_layout_passes=False),
)
def _sc_segsum(src_hbm, dst_hbm, feat_hbm, out_sum, out_deg,
               sidx_v, didx_v, rows_v, tsidx_v, tdidx_v, trows_v,
               deg_v, acc_s, gsem, isem, ssem):
    cid = lax.axis_index("c")
    sid = lax.axis_index("s")
    wid = cid * NS + sid

    zero16 = jnp.zeros((16,), jnp.float32)
    one16 = jnp.ones((16,), jnp.float32)

    stage_v = rows_v[0].at[pl.ds(0, ZR)]

    def _fill(i, carry):
        for c in range(D // 16):
            rows_v[0][i, pl.ds(c * 16, 16)] = zero16
        return carry
    lax.fori_loop(0, ZR, _fill, 0)

    def _zdeg(k, carry):
        deg_v[pl.ds(k * 16, 16)] = zero16
        return carry
    lax.fori_loop(0, NP // 16, _zdeg, 0)

    def _zero(j, carry):
        pltpu.sync_copy(stage_v, acc_s.at[pl.ds(sid * RPS + j * ZR, ZR)])
        return carry
    lax.fori_loop(0, RPS // ZR, _zero, 0)

    plsc.subcore_barrier()

    def _wait_scatter(b):
        pltpu.make_async_copy(rows_v[b], acc_s.at[didx_v[b]], ssem[b]).wait()

    def _burst(c0, waits):
        for b in waits:
            _wait_scatter(b)
        idesc = []
        for b in range(NB):
            base = wid * EPW + (c0 + b) * C
            idesc.append((
                pltpu.async_copy(src_hbm.at[pl.ds(base, C)], sidx_v[b], isem[b]),
                pltpu.async_copy(dst_hbm.at[pl.ds(base, C)], didx_v[b], isem[b]),
            ))
        gdesc = []
        for b in range(NB):
            idesc[b][0].wait()
            idesc[b][1].wait()
            gdesc.append(pltpu.async_copy(
                feat_hbm.at[sidx_v[b]], rows_v[b], gsem[b]))
        for b in range(NB):
            gdesc[b].wait()
            for k in range(C // 16):
                idx16 = didx_v[b][pl.ds(k * 16, 16)]
                plsc.addupdate_scatter(deg_v, [idx16], one16)
            pltpu.async_copy(rows_v[b], acc_s.at[didx_v[b]], ssem[b], add=True)

    _burst(0, [])

    def _step(j, carry):
        _burst(NB * (j + 1), range(NB))
        return carry
    lax.fori_loop(0, NSTEP - 1, _step, 0)

    tbase = wid * EPW + NFULL * C
    pltpu.sync_copy(src_hbm.at[pl.ds(tbase, CT)], tsidx_v)
    pltpu.sync_copy(dst_hbm.at[pl.ds(tbase, CT)], tdidx_v)
    pltpu.async_copy(feat_hbm.at[tsidx_v], trows_v, gsem[0]).wait()
    for k in range(CT // 16):
        idx16 = tdidx_v[pl.ds(k * 16, 16)]
        plsc.addupdate_scatter(deg_v, [idx16], one16)
    pltpu.sync_copy(trows_v, acc_s.at[tdidx_v], add=True)
    for b in range(NB):
        _wait_scatter(b)

    plsc.subcore_barrier()

    def _writeout(j, carry):
        row = sid * RPS + j * ZR
        pltpu.sync_copy(acc_s.at[pl.ds(row, ZR)], stage_v)
        pltpu.sync_copy(stage_v, out_sum.at[pl.ds(cid * NP + row, ZR)])
        return carry
    lax.fori_loop(0, RPS // ZR, _writeout, 0)
    pltpu.sync_copy(deg_v, out_deg.at[pl.ds(wid * NP, NP)])


BLK = 640
BLK_GRID = NP // BLK


def _tc_body(feat_r, ps_r, pd_r, wn_r, ws_r, b_r, out_r):
    s = ps_r[0] + ps_r[1]
    d = jnp.sum(pd_r[...], axis=0)[:, None]
    h = s / jnp.maximum(d, 1.0)
    a = lax.dot_general(feat_r[...], ws_r[...], (((1,), (1,)), ((), ())),
                        preferred_element_type=jnp.float32)
    m = lax.dot_general(h, wn_r[...], (((1,), (1,)), ((), ())),
                        preferred_element_type=jnp.float32)
    out_r[...] = a + m + b_r[0]


def kernel(feat, edge_index, W_neigh, W_self, b_self):
    src = edge_index[0].astype(jnp.int32)
    dst = edge_index[1].astype(jnp.int32)

    psum_flat, pdeg_flat = _sc_segsum(src, dst, feat)
    psum = psum_flat.reshape(NC, NP, D)
    pdeg = pdeg_flat.reshape(NW, NP)

    out = pl.pallas_call(
        _tc_body,
        grid=(BLK_GRID,),
        in_specs=[
            pl.BlockSpec((BLK, D), lambda i: (i, 0)),
            pl.BlockSpec((NC, BLK, D), lambda i: (0, i, 0)),
            pl.BlockSpec((NW, BLK), lambda i: (0, i)),
            pl.BlockSpec((D, D), lambda i: (0, 0)),
            pl.BlockSpec((D, D), lambda i: (0, 0)),
            pl.BlockSpec((1, D), lambda i: (0, 0)),
        ],
        out_specs=pl.BlockSpec((BLK, D), lambda i: (i, 0)),
        out_shape=jax.ShapeDtypeStruct((N, D), jnp.float32),
    )(feat, psum, pdeg, W_neigh, W_self, b_self.reshape(1, D))
    return out


# speedup vs baseline: 10.3277x; 1.0025x over previous
import functools

import jax
import jax.numpy as jnp
from jax import lax
from jax.experimental import pallas as pl
from jax.experimental.pallas import tpu as pltpu
from jax.experimental.pallas import tpu_sc as plsc

N = 10000
E = 320000
D = 128
NP = 10240

NC = 2
NS = 16
NW = NC * NS
EPW = E // NW
C = 128
NB = 2
NFULL = EPW // C
NSTEP = NFULL // NB
CT = EPW - NFULL * C
RPS = NP // NS
ZR = 16

_mesh = plsc.VectorSubcoreMesh(core_axis_name="c", subcore_axis_name="s")


@functools.partial(
    pl.kernel,
    mesh=_mesh,
    out_type=[
        jax.ShapeDtypeStruct((NC * NP, D), jnp.float32),
        jax.ShapeDtypeStruct((NW * NP,), jnp.float32),
    ],
    scratch_types=[
        [pltpu.VMEM((C,), jnp.int32) for _ in range(NB)],
        [pltpu.VMEM((C,), jnp.int32) for _ in range(NB)],
        [pltpu.VMEM((C, D), jnp.float32) for _ in range(NB)],
        pltpu.VMEM((CT,), jnp.int32),
        pltpu.VMEM((CT,), jnp.int32),
        pltpu.VMEM((CT, D), jnp.float32),
        pltpu.VMEM((NP,), jnp.float32),
        pltpu.VMEM_SHARED((NP, D), jnp.float32),
        [pltpu.SemaphoreType.DMA for _ in range(NB)],
        [pltpu.SemaphoreType.DMA for _ in range(NB)],
        [pltpu.SemaphoreType.DMA for _ in range(NB)],
    ],
    compiler_params=pltpu.CompilerParams(needs_layout_passes=False),
)
def _sc_segsum(src_hbm, dst_hbm, feat_hbm, out_sum, out_deg,
               sidx_v, didx_v, rows_v, tsidx_v, tdidx_v, trows_v,
               deg_v, acc_s, gsem, isem, ssem):
    cid = lax.axis_index("c")
    sid = lax.axis_index("s")
    wid = cid * NS + sid

    zero16 = jnp.zeros((16,), jnp.float32)
    one16 = jnp.ones((16,), jnp.float32)

    stage_v = rows_v[0].at[pl.ds(0, ZR)]

    def _fill(i, carry):
        for c in range(D // 16):
            rows_v[0][i, pl.ds(c * 16, 16)] = zero16
        return carry
    lax.fori_loop(0, ZR, _fill, 0)

    def _zdeg(k, carry):
        deg_v[pl.ds(k * 16, 16)] = zero16
        return carry
    lax.fori_loop(0, NP // 16, _zdeg, 0)

    def _zero(j, carry):
        pltpu.sync_copy(stage_v, acc_s.at[pl.ds(sid * RPS + j * ZR, ZR)])
        return carry
    lax.fori_loop(0, RPS // ZR, _zero, 0)

    plsc.subcore_barrier()

    def _wait_scatter(b):
        pltpu.make_async_copy(rows_v[b], acc_s.at[didx_v[b]], ssem[b]).wait()

    def _burst(c0, waits):
        for b in waits:
            _wait_scatter(b)
        idesc = []
        for b in range(NB):
            base = wid * EPW + (c0 + b) * C
            idesc.append((
                pltpu.async_copy(src_hbm.at[pl.ds(base, C)], sidx_v[b], isem[b]),
                pltpu.async_copy(dst_hbm.at[pl.ds(base, C)], didx_v[b], isem[b]),
            ))
        gdesc = []
        for b in range(NB):
            idesc[b][0].wait()
            idesc[b][1].wait()
            gdesc.append(pltpu.async_copy(
                feat_hbm.at[sidx_v[b]], rows_v[b], gsem[b]))
        for b in range(NB):
            gdesc[b].wait()
            for k in range(C // 16):
                idx16 = didx_v[b][pl.ds(k * 16, 16)]
                plsc.addupdate_scatter(deg_v, [idx16], one16)
            pltpu.async_copy(rows_v[b], acc_s.at[didx_v[b]], ssem[b], add=True)

    _burst(0, [])

    def _step(j, carry):
        _burst(NB * (j + 1), range(NB))
        return carry
    lax.fori_loop(0, NSTEP - 1, _step, 0)

    tbase = wid * EPW + NFULL * C
    pltpu.sync_copy(src_hbm.at[pl.ds(tbase, CT)], tsidx_v)
    pltpu.sync_copy(dst_hbm.at[pl.ds(tbase, CT)], tdidx_v)
    pltpu.async_copy(feat_hbm.at[tsidx_v], trows_v, gsem[0]).wait()
    for k in range(CT // 16):
        idx16 = tdidx_v[pl.ds(k * 16, 16)]
        plsc.addupdate_scatter(deg_v, [idx16], one16)
    pltpu.sync_copy(trows_v, acc_s.at[tdidx_v], add=True)
    for b in range(NB):
        _wait_scatter(b)

    plsc.subcore_barrier()

    def _writeout(j, carry):
        row = sid * RPS + j * ZR
        pltpu.sync_copy(acc_s.at[pl.ds(row, ZR)], stage_v)
        pltpu.sync_copy(stage_v, out_sum.at[pl.ds(cid * NP + row, ZR)])
        return carry
    lax.fori_loop(0, RPS // ZR, _writeout, 0)
    pltpu.sync_copy(deg_v, out_deg.at[pl.ds(wid * NP, NP)])


BLK = 640
BLK_GRID = NP // BLK


def _tc_self_body(feat_r, ws_r, b_r, out_r):
    out_r[...] = lax.dot_general(
        feat_r[...], ws_r[...], (((1,), (1,)), ((), ())),
        preferred_element_type=jnp.float32) + b_r[0]


def _tc_comb_body(self_r, ps_r, pd_r, wn_r, out_r):
    s = ps_r[0] + ps_r[1]
    d = jnp.sum(pd_r[...], axis=0)[:, None]
    h = s / jnp.maximum(d, 1.0)
    m = lax.dot_general(h, wn_r[...], (((1,), (1,)), ((), ())),
                        preferred_element_type=jnp.float32)
    out_r[...] = self_r[...] + m


def kernel(feat, edge_index, W_neigh, W_self, b_self):
    src = edge_index[0].astype(jnp.int32)
    dst = edge_index[1].astype(jnp.int32)

    psum_flat, pdeg_flat = _sc_segsum(src, dst, feat)
    psum = psum_flat.reshape(NC, NP, D)
    pdeg = pdeg_flat.reshape(NW, NP)

    self_out = pl.pallas_call(
        _tc_self_body,
        grid=(BLK_GRID,),
        in_specs=[
            pl.BlockSpec((BLK, D), lambda i: (i, 0)),
            pl.BlockSpec((D, D), lambda i: (0, 0)),
            pl.BlockSpec((1, D), lambda i: (0, 0)),
        ],
        out_specs=pl.BlockSpec((BLK, D), lambda i: (i, 0)),
        out_shape=jax.ShapeDtypeStruct((N, D), jnp.float32),
    )(feat, W_self, b_self.reshape(1, D))

    out = pl.pallas_call(
        _tc_comb_body,
        grid=(BLK_GRID,),
        in_specs=[
            pl.BlockSpec((BLK, D), lambda i: (i, 0)),
            pl.BlockSpec((NC, BLK, D), lambda i: (0, i, 0)),
            pl.BlockSpec((NW, BLK), lambda i: (0, i)),
            pl.BlockSpec((D, D), lambda i: (0, 0)),
        ],
        out_specs=pl.BlockSpec((BLK, D), lambda i: (i, 0)),
        out_shape=jax.ShapeDtypeStruct((N, D), jnp.float32),
    )(self_out, psum, pdeg, W_neigh)
    return out


# speedup vs baseline: 10.5546x; 1.0220x over previous
import functools

import jax
import jax.numpy as jnp
from jax import lax
from jax.experimental import pallas as pl
from jax.experimental.pallas import tpu as pltpu
from jax.experimental.pallas import tpu_sc as plsc

N = 10000
E = 320000
D = 128
NP = 10240

NC = 2
NS = 16
NW = NC * NS
EPW = E // NW
C = 128
NB = 2
NFULL = EPW // C
NSTEP = NFULL // NB
CT = EPW - NFULL * C
RPS = NP // NS
ZR = 16

_mesh = plsc.VectorSubcoreMesh(core_axis_name="c", subcore_axis_name="s")


@functools.partial(
    pl.kernel,
    mesh=_mesh,
    out_type=[
        jax.ShapeDtypeStruct((NC * NP, D), jnp.float32),
        jax.ShapeDtypeStruct((NW * NP,), jnp.float32),
    ],
    scratch_types=[
        [pltpu.VMEM((C,), jnp.int32) for _ in range(NB)],
        [pltpu.VMEM((C,), jnp.int32) for _ in range(NB)],
        [pltpu.VMEM((C, D), jnp.float32) for _ in range(NB)],
        pltpu.VMEM((CT,), jnp.int32),
        pltpu.VMEM((CT,), jnp.int32),
        pltpu.VMEM((CT, D), jnp.float32),
        pltpu.VMEM((NP,), jnp.float32),
        pltpu.VMEM_SHARED((NP, D), jnp.float32),
        [pltpu.SemaphoreType.DMA for _ in range(NB)],
        [pltpu.SemaphoreType.DMA for _ in range(NB)],
        [pltpu.SemaphoreType.DMA for _ in range(NB)],
    ],
    compiler_params=pltpu.CompilerParams(needs_layout_passes=False),
)
def _sc_segsum(src_hbm, dst_hbm, feat_hbm, out_sum, out_deg,
               sidx_v, didx_v, rows_v, tsidx_v, tdidx_v, trows_v,
               deg_v, acc_s, gsem, isem, ssem):
    cid = lax.axis_index("c")
    sid = lax.axis_index("s")
    wid = cid * NS + sid

    zero16 = jnp.zeros((16,), jnp.float32)
    one16 = jnp.ones((16,), jnp.float32)

    stage_v = rows_v[0].at[pl.ds(0, ZR)]

    def _fill(i, carry):
        for c in range(D // 16):
            rows_v[0][i, pl.ds(c * 16, 16)] = zero16
        return carry
    lax.fori_loop(0, ZR, _fill, 0)

    def _zdeg(k, carry):
        deg_v[pl.ds(k * 16, 16)] = zero16
        return carry
    lax.fori_loop(0, NP // 16, _zdeg, 0)

    def _zero(j, carry):
        pltpu.sync_copy(stage_v, acc_s.at[pl.ds(sid * RPS + j * ZR, ZR)])
        return carry
    lax.fori_loop(0, RPS // ZR, _zero, 0)

    plsc.subcore_barrier()

    def _wait_scatter(b):
        pltpu.make_async_copy(rows_v[b], acc_s.at[didx_v[b]], ssem[b]).wait()

    def _burst(c0, waits):
        for b in waits:
            _wait_scatter(b)
        idesc = []
        for b in range(NB):
            base = wid * EPW + (c0 + b) * C
            idesc.append((
                pltpu.async_copy(src_hbm.at[pl.ds(base, C)], sidx_v[b], isem[b]),
                pltpu.async_copy(dst_hbm.at[pl.ds(base, C)], didx_v[b], isem[b]),
            ))
        gdesc = []
        for b in range(NB):
            idesc[b][0].wait()
            idesc[b][1].wait()
            gdesc.append(pltpu.async_copy(
                feat_hbm.at[sidx_v[b]], rows_v[b], gsem[b]))
        for b in range(NB):
            gdesc[b].wait()
            for k in range(C // 16):
                idx16 = didx_v[b][pl.ds(k * 16, 16)]
                plsc.addupdate_scatter(deg_v, [idx16], one16)
            pltpu.async_copy(rows_v[b], acc_s.at[didx_v[b]], ssem[b], add=True)

    _burst(0, [])

    def _step(j, carry):
        _burst(NB * (j + 1), range(NB))
        return carry
    lax.fori_loop(0, NSTEP - 1, _step, 0)

    tbase = wid * EPW + NFULL * C
    pltpu.sync_copy(src_hbm.at[pl.ds(tbase, CT)], tsidx_v)
    pltpu.sync_copy(dst_hbm.at[pl.ds(tbase, CT)], tdidx_v)
    pltpu.async_copy(feat_hbm.at[tsidx_v], trows_v, gsem[0]).wait()
    for k in range(CT // 16):
        idx16 = tdidx_v[pl.ds(k * 16, 16)]
        plsc.addupdate_scatter(deg_v, [idx16], one16)
    pltpu.sync_copy(trows_v, acc_s.at[tdidx_v], add=True)
    for b in range(NB):
        _wait_scatter(b)

    plsc.subcore_barrier()

    row0 = sid * RPS
    pltpu.sync_copy(acc_s.at[pl.ds(row0, RPS)],
                    out_sum.at[pl.ds(cid * NP + row0, RPS)])
    pltpu.sync_copy(deg_v, out_deg.at[pl.ds(wid * NP, NP)])


BLK = 640
BLK_GRID = NP // BLK


def _tc_self_body(feat_r, ws_r, b_r, out_r):
    out_r[...] = lax.dot_general(
        feat_r[...], ws_r[...], (((1,), (1,)), ((), ())),
        preferred_element_type=jnp.float32) + b_r[0]


def _tc_comb_body(self_r, ps_r, pd_r, wn_r, out_r):
    s = ps_r[0] + ps_r[1]
    d = jnp.sum(pd_r[...], axis=0)[:, None]
    h = s / jnp.maximum(d, 1.0)
    m = lax.dot_general(h, wn_r[...], (((1,), (1,)), ((), ())),
                        preferred_element_type=jnp.float32)
    out_r[...] = self_r[...] + m


def kernel(feat, edge_index, W_neigh, W_self, b_self):
    src = edge_index[0].astype(jnp.int32)
    dst = edge_index[1].astype(jnp.int32)

    psum_flat, pdeg_flat = _sc_segsum(src, dst, feat)
    psum = psum_flat.reshape(NC, NP, D)
    pdeg = pdeg_flat.reshape(NW, NP)

    self_out = pl.pallas_call(
        _tc_self_body,
        grid=(BLK_GRID,),
        in_specs=[
            pl.BlockSpec((BLK, D), lambda i: (i, 0)),
            pl.BlockSpec((D, D), lambda i: (0, 0)),
            pl.BlockSpec((1, D), lambda i: (0, 0)),
        ],
        out_specs=pl.BlockSpec((BLK, D), lambda i: (i, 0)),
        out_shape=jax.ShapeDtypeStruct((N, D), jnp.float32),
    )(feat, W_self, b_self.reshape(1, D))

    out = pl.pallas_call(
        _tc_comb_body,
        grid=(BLK_GRID,),
        in_specs=[
            pl.BlockSpec((BLK, D), lambda i: (i, 0)),
            pl.BlockSpec((NC, BLK, D), lambda i: (0, i, 0)),
            pl.BlockSpec((NW, BLK), lambda i: (0, i)),
            pl.BlockSpec((D, D), lambda i: (0, 0)),
        ],
        out_specs=pl.BlockSpec((BLK, D), lambda i: (i, 0)),
        out_shape=jax.ShapeDtypeStruct((N, D), jnp.float32),
    )(self_out, psum, pdeg, W_neigh)
    return out
